# baseline (speedup 1.0000x reference)
import jax
import jax.numpy as jnp
from jax.experimental import pallas as pl
from jax.experimental.pallas import tpu as pltpu

R = 5000
NB = 3
W = 4


def _mlp_kernel(h_hbm, w1_ref, b1_ref, w2_ref, b2_ref, out_hbm,
                h_buf, o_buf, in_sem, out_sem):
    n2 = h_hbm.shape[0]
    nch = n2 // R

    def in_copy(chunk, slot):
        return pltpu.make_async_copy(
            h_hbm.at[pl.ds(chunk * R, R), :], h_buf.at[slot], in_sem.at[slot])

    def out_copy(chunk, slot):
        return pltpu.make_async_copy(
            o_buf.at[slot], out_hbm.at[pl.ds(chunk * R, R), :],
            out_sem.at[slot])

    for j in range(NB - 1):
        in_copy(j, j).start()

    for chunk in range(nch):
        s = chunk % NB
        in_copy(chunk, s).wait()
        if chunk >= NB:
            out_copy(chunk - NB, s).wait()

        for c in range(W):
            hs = h_buf[s, :, 128 * c:128 * (c + 1)]
            x = jnp.dot(hs, w1_ref[...], preferred_element_type=jnp.float32)
            x = jnp.maximum(x + b1_ref[...], 0.0)
            o_buf[s, :, 64 * c:64 * (c + 1)] = (
                jnp.dot(x, w2_ref[...], preferred_element_type=jnp.float32)
                + b2_ref[...])

        out_copy(chunk, s).start()

        nxt = chunk + NB - 1
        if nxt < nch:
            in_copy(nxt, nxt % NB).start()

    for j in range(max(nch - NB, 0), nch):
        out_copy(j, j % NB).wait()


def kernel(h, W1, b1, W2, b2):
    N, IN = h.shape
    HID = W1.shape[1]
    OUT = W2.shape[1]
    n2 = N // W
    assert n2 % R == 0
    h2 = h.reshape(n2, W * IN)
    b1r = b1.reshape(1, HID)
    b2r = b2.reshape(1, OUT)
    out2 = pl.pallas_call(
        _mlp_kernel,
        in_specs=[
            pl.BlockSpec(memory_space=pltpu.MemorySpace.HBM),
            pl.BlockSpec(memory_space=pltpu.MemorySpace.VMEM),
            pl.BlockSpec(memory_space=pltpu.MemorySpace.VMEM),
            pl.BlockSpec(memory_space=pltpu.MemorySpace.VMEM),
            pl.BlockSpec(memory_space=pltpu.MemorySpace.VMEM),
        ],
        out_specs=pl.BlockSpec(memory_space=pltpu.MemorySpace.HBM),
        out_shape=jax.ShapeDtypeStruct((n2, W * OUT), jnp.float32),
        scratch_shapes=[
            pltpu.VMEM((NB, R, W * IN), jnp.float32),
            pltpu.VMEM((NB, R, W * OUT), jnp.float32),
            pltpu.SemaphoreType.DMA((NB,)),
            pltpu.SemaphoreType.DMA((NB,)),
        ],
    )(h2, W1, b1r, W2, b2r)
    return out2.reshape(N, OUT)


# speedup vs baseline: 2.3802x; 2.3802x over previous
import jax
import jax.numpy as jnp
from jax.experimental import pallas as pl
from jax.experimental.pallas import tpu as pltpu

C = 4000
NB = 5


def _mlp_kernel(h_hbm, w1_ref, b1_ref, w2_ref, b2_ref, out_hbm, *scratch):
    h_bufs = scratch[:NB]
    o_bufs = scratch[NB:2 * NB]
    in_sems = scratch[2 * NB:3 * NB]
    out_sems = scratch[3 * NB:4 * NB]
    n = h_hbm.shape[0]
    nch = n // C

    def in_copy(chunk, slot):
        return pltpu.make_async_copy(
            h_hbm.at[pl.ds(chunk * C, C), :], h_bufs[slot], in_sems[slot])

    def out_copy(chunk, slot):
        return pltpu.make_async_copy(
            o_bufs[slot], out_hbm.at[pl.ds(chunk * C, C), :], out_sems[slot])

    for j in range(NB - 1):
        in_copy(j, j).start()

    for chunk in range(nch):
        s = chunk % NB
        in_copy(chunk, s).wait()
        if chunk >= NB:
            out_copy(chunk - NB, s).wait()

        x = jnp.dot(h_bufs[s][...], w1_ref[...],
                    preferred_element_type=jnp.float32)
        x = jnp.maximum(x + b1_ref[...], 0.0)
        o_bufs[s][...] = jnp.dot(x, w2_ref[...],
                                 preferred_element_type=jnp.float32) + b2_ref[...]

        out_copy(chunk, s).start()

        nxt = chunk + NB - 1
        if nxt < nch:
            in_copy(nxt, nxt % NB).start()

    for j in range(max(nch - NB, 0), nch):
        out_copy(j, j % NB).wait()


def kernel(h, W1, b1, W2, b2):
    N, IN = h.shape
    HID = W1.shape[1]
    OUT = W2.shape[1]
    assert N % C == 0
    b1r = b1.reshape(1, HID)
    b2r = b2.reshape(1, OUT)
    return pl.pallas_call(
        _mlp_kernel,
        in_specs=[
            pl.BlockSpec(memory_space=pltpu.MemorySpace.HBM),
            pl.BlockSpec(memory_space=pltpu.MemorySpace.VMEM),
            pl.BlockSpec(memory_space=pltpu.MemorySpace.VMEM),
            pl.BlockSpec(memory_space=pltpu.MemorySpace.VMEM),
            pl.BlockSpec(memory_space=pltpu.MemorySpace.VMEM),
        ],
        out_specs=pl.BlockSpec(memory_space=pltpu.MemorySpace.HBM),
        out_shape=jax.ShapeDtypeStruct((N, OUT), jnp.float32),
        scratch_shapes=(
            [pltpu.VMEM((C, IN), jnp.float32) for _ in range(NB)]
            + [pltpu.VMEM((C, OUT), jnp.float32) for _ in range(NB)]
            + [pltpu.SemaphoreType.DMA for _ in range(NB)]
            + [pltpu.SemaphoreType.DMA for _ in range(NB)]
        ),
    )(h, W1, b1r, W2, b2r)
